# baseline (speedup 1.0000x reference)
import functools

import jax
import jax.numpy as jnp
from jax import lax
from jax.experimental import pallas as pl
from jax.experimental.pallas import tpu as pltpu
from jax.experimental.pallas import tpu_sc as plsc

B = 16384
MF_DIM = 16
MLP_HALF = 32
NC = 2
NS = 16
NW = NC * NS
BPW = B // NW
CH = 64
NCH = BPW // CH
L = 16

_mesh = plsc.VectorSubcoreMesh(core_axis_name="c", subcore_axis_name="s")


def _iota16():
    return lax.iota(jnp.int32, L)


@functools.partial(
    pl.kernel,
    mesh=_mesh,
    compiler_params=pltpu.CompilerParams(needs_layout_passes=False),
    out_type=jax.ShapeDtypeStruct((B, 128), jnp.float32),
    scratch_types=[
        pltpu.VMEM((BPW,), jnp.int32),
        pltpu.VMEM((BPW,), jnp.int32),
        pltpu.VMEM((BPW,), jnp.int32),
        pltpu.VMEM((BPW,), jnp.int32),
        pltpu.VMEM((BPW,), jnp.int32),
        pltpu.VMEM((BPW,), jnp.int32),
        pltpu.VMEM((CH, 128), jnp.float32),
        pltpu.VMEM((CH, 128), jnp.float32),
        pltpu.VMEM((CH, 128), jnp.float32),
        pltpu.VMEM((CH, 128), jnp.float32),
        pltpu.VMEM((CH, 128), jnp.float32),
        pltpu.SemaphoreType.DMA,
    ],
)
def _sc_gather(user_ids, item_ids, mlp_user, mlp_item, mf_user, mf_item,
               out, uidx, iidx, ur, ir, fur, fir,
               ubuf, ibuf, fubuf, fibuf, comp, sem):
    wid = lax.axis_index("s") * NC + lax.axis_index("c")
    base = wid * BPW
    pltpu.sync_copy(user_ids.at[pl.ds(base, BPW)], uidx)
    pltpu.sync_copy(item_ids.at[pl.ds(base, BPW)], iidx)
    for t in range(BPW // L):
        sl = pl.ds(t * L, L)
        u = uidx[sl]
        i = iidx[sl]
        ur[sl] = lax.shift_right_logical(u, 2)
        ir[sl] = lax.shift_right_logical(i, 2)
        fur[sl] = lax.shift_right_logical(u, 3)
        fir[sl] = lax.shift_right_logical(i, 3)

    def chunk_body(ch, _):
        cb = ch * CH
        csl = pl.ds(cb, CH)
        c0 = pltpu.async_copy(mlp_user.at[ur.at[csl]], ubuf, sem)
        c1 = pltpu.async_copy(mlp_item.at[ir.at[csl]], ibuf, sem)
        c2 = pltpu.async_copy(mf_user.at[fur.at[csl]], fubuf, sem)
        c3 = pltpu.async_copy(mf_item.at[fir.at[csl]], fibuf, sem)
        c0.wait()
        c1.wait()
        c2.wait()
        c3.wait()
        for blk in range(CH // L):
            lrow = blk * L + _iota16()
            uid = uidx[pl.ds(cb + blk * L, L)]
            iid = iidx[pl.ds(cb + blk * L, L)]
            offu = (uid & 3) * 32
            offi = (iid & 3) * 32
            offfu = (uid & 7) * 16
            offfi = (iid & 7) * 16
            for c in range(MLP_HALF):
                cv = jnp.full((L,), c, jnp.int32)
                plsc.store_scatter(comp, [lrow, cv],
                                   plsc.load_gather(ubuf, [lrow, offu + c]))
                plsc.store_scatter(comp, [lrow, cv + 32],
                                   plsc.load_gather(ibuf, [lrow, offi + c]))
            for c in range(MF_DIM):
                cv = jnp.full((L,), c, jnp.int32)
                plsc.store_scatter(comp, [lrow, cv + 64],
                                   plsc.load_gather(fubuf, [lrow, offfu + c]))
                plsc.store_scatter(comp, [lrow, cv + 80],
                                   plsc.load_gather(fibuf, [lrow, offfi + c]))
        pltpu.sync_copy(comp, out.at[pl.ds(base + cb, CH)])
        return _

    lax.fori_loop(0, NCH, chunk_body, 0)


BT = 2048


def _tc_body(p_ref, W1_ref, b1_ref, W2_ref, b2_ref,
             W3_ref, b3_ref, W4_ref, b4_ref, out_ref):
    x = p_ref[:, 0:64]
    h = jnp.maximum(jnp.dot(x, W1_ref[...], preferred_element_type=jnp.float32)
                    + b1_ref[...], 0.0)
    h = jnp.maximum(jnp.dot(h, W2_ref[...], preferred_element_type=jnp.float32)
                    + b2_ref[...], 0.0)
    h = jnp.maximum(jnp.dot(h, W3_ref[...], preferred_element_type=jnp.float32)
                    + b3_ref[...], 0.0)
    mf = jnp.sum(p_ref[:, 64:80] * p_ref[:, 80:96], axis=1, keepdims=True)
    z = (mf * W4_ref[0:1, :]
         + jnp.dot(h, W4_ref[1:9, :], preferred_element_type=jnp.float32)
         + b4_ref[...])
    out_ref[...] = 1.0 / (1.0 + jnp.exp(-z))


def _tc_mlp(packed, W1, b1r, W2, b2r, W3, b3r, W4p, b4r):
    grid = (B // BT,)
    full = lambda g: (0, 0)
    tile = lambda g: (g, 0)
    return pl.pallas_call(
        _tc_body,
        grid=grid,
        in_specs=[
            pl.BlockSpec((BT, 128), tile),
            pl.BlockSpec((64, 32), full),
            pl.BlockSpec((1, 32), full),
            pl.BlockSpec((32, 16), full),
            pl.BlockSpec((1, 16), full),
            pl.BlockSpec((16, 8), full),
            pl.BlockSpec((1, 8), full),
            pl.BlockSpec((16, 1), full),
            pl.BlockSpec((1, 1), full),
        ],
        out_specs=pl.BlockSpec((BT, 1), tile),
        out_shape=jax.ShapeDtypeStruct((B, 1), jnp.float32),
    )(packed, W1, b1r, W2, b2r, W3, b3r, W4p, b4r)


def kernel(user_ids, item_ids, mf_user, mf_item, mlp_user, mlp_item,
           W1, b1, W2, b2, W3, b3, W4, b4):
    packed = _sc_gather(user_ids, item_ids,
                        mlp_user.reshape(-1, 128), mlp_item.reshape(-1, 128),
                        mf_user.reshape(-1, 128), mf_item.reshape(-1, 128))
    W4p = jnp.pad(W4, ((0, 7), (0, 0)))
    return _tc_mlp(packed, W1, b1.reshape(1, 32), W2, b2.reshape(1, 16),
                   W3, b3.reshape(1, 8), W4p, b4.reshape(1, 1))
